# baseline (speedup 1.0000x reference)
import functools

import jax
import jax.numpy as jnp
from jax import lax
from jax.experimental import pallas as pl
from jax.experimental.pallas import tpu as pltpu
from jax.experimental.pallas import tpu_sc as plsc

BATCH = 16384
DIM = 64
NUSERS = 1000000
NC = 2
NS = 16
NW = NC * NS
CHW = 256
NQ = (NUSERS + CHW - 1) // CHW
Q_PER_W = (NQ + NW - 1) // NW
TAILQ = NQ - 1
TAILW = NUSERS - TAILQ * CHW

_mesh = plsc.VectorSubcoreMesh(core_axis_name="c", subcore_axis_name="s")


@functools.partial(
    pl.kernel,
    mesh=_mesh,
    out_type=jax.ShapeDtypeStruct((BATCH, DIM), jnp.float32),
    scratch_types=[
        pltpu.VMEM((BATCH,), jnp.int32),
        pltpu.VMEM((BATCH + 16,), jnp.int32),
        pltpu.VMEM((BATCH + 16,), jnp.int32),
        pltpu.VMEM((2, DIM, CHW), jnp.float32),
        pltpu.VMEM((272,), jnp.int32),
        pltpu.VMEM((272,), jnp.int32),
        pltpu.VMEM((32, DIM), jnp.float32),
        pltpu.VMEM((TAILW, DIM), jnp.float32),
        pltpu.SemaphoreType.DMA,
        pltpu.SemaphoreType.DMA,
    ],
    compiler_params=pltpu.CompilerParams(needs_layout_passes=False),
)
def _gather_kernel(idx_hbm, table_t_hbm, tail_hbm, out_hbm, ids_v, hid_v,
                   hpos_v, chunk_v, cbuf_v, pbuf_v, stage_v, tail_v,
                   sem_c, sem_o):
    wid = lax.axis_index("s") * NC + lax.axis_index("c")
    lanes = lax.iota(jnp.int32, 16)
    q_base = wid * Q_PER_W
    lo = q_base * CHW
    hi = jnp.minimum(lo + Q_PER_W * CHW, NUSERS)

    pltpu.sync_copy(idx_hbm, ids_v)
    pltpu.sync_copy(tail_hbm, tail_v)

    def scan_body(t, off):
        v = ids_v[pl.ds(t * 16, 16)]
        m = (v >= lo) & (v < hi)
        plsc.store_compressed(hid_v.at[pl.ds(off, 16)], v, mask=m)
        plsc.store_compressed(hpos_v.at[pl.ds(off, 16)], t * 16 + lanes, mask=m)
        return off + plsc.all_reduce_population_count(m)[0]

    nh = lax.fori_loop(0, BATCH // 16, scan_body, jnp.int32(0))

    def fire(k):
        q = q_base + k

        @pl.when(q < TAILQ)
        def _():
            g0 = pl.multiple_of(q * CHW, 128)
            pltpu.async_copy(
                table_t_hbm.at[:, pl.ds(g0, CHW)], chunk_v.at[k & 1], sem_c
            )

    def wait_chunk(k):
        q = q_base + k

        @pl.when(q < TAILQ)
        def _():
            pltpu.make_async_copy(
                table_t_hbm.at[:, pl.ds(0, CHW)], chunk_v.at[0], sem_c
            ).wait()

    def drain_out_rows(n):
        def d(_, carry):
            pltpu.make_async_copy(
                stage_v.at[pl.ds(0, 1)], out_hbm.at[pl.ds(0, 1)], sem_o
            ).wait()
            return carry

        lax.fori_loop(0, n, d, jnp.int32(0))

    def extract_vec(e, nc, k, fcnt, dcnt):
        rem = jnp.minimum(nc - e * 16, 16)
        ndrain = jnp.maximum(fcnt + rem - dcnt - 32, 0)
        drain_out_rows(ndrain)
        cv = cbuf_v[pl.ds(e * 16, 16)]
        pv = pbuf_v[pl.ds(e * 16, 16)]
        for lane in range(16):
            @pl.when(lane < rem)
            def _():
                c = cv[lane]
                p = pv[lane]
                slot = (fcnt + lane) & 31
                row = stage_v.at[slot]
                for g in range(DIM // 16):
                    vals = plsc.load_gather(
                        chunk_v,
                        [jnp.full((16,), k & 1, jnp.int32),
                         lanes + (16 * g),
                         jnp.full((16,), c, jnp.int32)],
                    )
                    row[pl.ds(16 * g, 16)] = vals
                pltpu.async_copy(
                    stage_v.at[pl.ds(slot, 1)], out_hbm.at[pl.ds(p, 1)], sem_o
                )
        return fcnt + rem, dcnt + ndrain

    def extract_tail_vec(e, nc, fcnt, dcnt):
        rem = jnp.minimum(nc - e * 16, 16)
        cv = cbuf_v[pl.ds(e * 16, 16)]
        pv = pbuf_v[pl.ds(e * 16, 16)]
        for lane in range(16):
            @pl.when(lane < rem)
            def _():
                c = cv[lane]
                p = pv[lane]
                pltpu.async_copy(
                    tail_v.at[pl.ds(c, 1)], out_hbm.at[pl.ds(p, 1)], sem_o
                )
        return fcnt + rem, dcnt

    def chunk_body(k, st):
        @pl.when(k + 1 < Q_PER_W)
        def _():
            fire(k + 1)

        wait_chunk(k)
        q = q_base + k
        g0 = q * CHW

        def batch_body(t2, st2):
            def gather_hits(t, off):
                hv = hid_v[pl.ds(t * 16, 16)]
                valid = (t * 16 + lanes) < nh
                m = valid & (hv >= g0) & (hv < g0 + CHW)
                pc = plsc.all_reduce_population_count(m)[0]

                @pl.when(pc > 0)
                def _():
                    pvv = hpos_v[pl.ds(t * 16, 16)]
                    plsc.store_compressed(
                        cbuf_v.at[pl.ds(off, 16)], hv - g0, mask=m)
                    plsc.store_compressed(
                        pbuf_v.at[pl.ds(off, 16)], pvv, mask=m)

                return off + pc

            nc = lax.fori_loop(
                t2 * 16, jnp.minimum(t2 * 16 + 16, (nh + 15) // 16),
                gather_hits, jnp.int32(0))

            def ex(e, st3):
                return lax.cond(
                    q == TAILQ,
                    lambda s: extract_tail_vec(e, nc, s[0], s[1]),
                    lambda s: extract_vec(e, nc, k, s[0], s[1]),
                    st3,
                )

            return lax.fori_loop(0, (nc + 15) // 16, ex, st2)

        nb = jnp.where(q < NQ, (nh + 255) // 256, 0)
        return lax.fori_loop(0, nb, batch_body, st)

    fire(0)
    fcnt, dcnt = lax.fori_loop(
        0, Q_PER_W, chunk_body, (jnp.int32(0), jnp.int32(0)))
    drain_out_rows(fcnt - dcnt)


def kernel(user_ids, long_pref_emb):
    tail = long_pref_emb[NUSERS - TAILW:]
    return _gather_kernel(user_ids.astype(jnp.int32), long_pref_emb.T, tail)


# speedup vs baseline: 1.0886x; 1.0886x over previous
import functools

import jax
import jax.numpy as jnp
from jax import lax
from jax.experimental import pallas as pl
from jax.experimental.pallas import tpu as pltpu
from jax.experimental.pallas import tpu_sc as plsc

BATCH = 16384
DIM = 64
NUSERS = 1000000
NC = 2
NS = 16
NW = NC * NS
CHW = 256
NQ = (NUSERS + CHW - 1) // CHW
Q_PER_W = (NQ + NW - 1) // NW
TAILQ = NQ - 1
TAILW = NUSERS - TAILQ * CHW

_mesh = plsc.VectorSubcoreMesh(core_axis_name="c", subcore_axis_name="s")


@functools.partial(
    pl.kernel,
    mesh=_mesh,
    out_type=jax.ShapeDtypeStruct((BATCH, DIM), jnp.float32),
    scratch_types=[
        pltpu.VMEM((BATCH,), jnp.int32),
        pltpu.VMEM((BATCH + 16,), jnp.int32),
        pltpu.VMEM((BATCH + 16,), jnp.int32),
        pltpu.VMEM((2, DIM, CHW), jnp.float32),
        pltpu.VMEM((272,), jnp.int32),
        pltpu.VMEM((272,), jnp.int32),
        pltpu.VMEM((32, DIM), jnp.float32),
        pltpu.VMEM((TAILW, DIM), jnp.float32),
        pltpu.SemaphoreType.DMA,
        pltpu.SemaphoreType.DMA,
    ],
    compiler_params=pltpu.CompilerParams(needs_layout_passes=False),
)
def _gather_kernel(idx_hbm, table_t_hbm, tail_hbm, out_hbm, ids_v, hid_v,
                   hpos_v, chunk_v, cbuf_v, pbuf_v, stage_v, tail_v,
                   sem_c, sem_o):
    wid = lax.axis_index("s") * NC + lax.axis_index("c")
    lanes = lax.iota(jnp.int32, 16)
    q_base = wid * Q_PER_W
    lo = q_base * CHW
    hi = jnp.minimum(lo + Q_PER_W * CHW, NUSERS)

    pltpu.sync_copy(idx_hbm, ids_v)
    pltpu.sync_copy(tail_hbm, tail_v)

    def scan_body(t, off):
        v = ids_v[pl.ds(t * 16, 16)]
        m = (v >= lo) & (v < hi)
        plsc.store_compressed(hid_v.at[pl.ds(off, 16)], v, mask=m)
        plsc.store_compressed(hpos_v.at[pl.ds(off, 16)], t * 16 + lanes, mask=m)
        return off + plsc.all_reduce_population_count(m)[0]

    nh = lax.fori_loop(0, BATCH // 16, scan_body, jnp.int32(0))

    def fire(k):
        q = q_base + k

        @pl.when(q < TAILQ)
        def _():
            g0 = pl.multiple_of(q * CHW, 128)
            pltpu.async_copy(
                table_t_hbm.at[:, pl.ds(g0, CHW)], chunk_v.at[k & 1], sem_c
            )

    def wait_chunk(k):
        q = q_base + k

        @pl.when(q < TAILQ)
        def _():
            pltpu.make_async_copy(
                table_t_hbm.at[:, pl.ds(0, CHW)], chunk_v.at[0], sem_c
            ).wait()

    def drain_out_rows(n):
        def d(_, carry):
            pltpu.make_async_copy(
                stage_v.at[pl.ds(0, 1)], out_hbm.at[pl.ds(0, 1)], sem_o
            ).wait()
            return carry

        lax.fori_loop(0, n, d, jnp.int32(0))

    def extract_vec(e, nc, k, fcnt, dcnt):
        rem = jnp.minimum(nc - e * 16, 16)
        ndrain = jnp.maximum(fcnt + rem - dcnt - 32, 0)
        drain_out_rows(ndrain)
        cv = cbuf_v[pl.ds(e * 16, 16)]
        pv = pbuf_v[pl.ds(e * 16, 16)]
        for lane in range(16):
            @pl.when(lane < rem)
            def _():
                c = cv[lane]
                p = pv[lane]
                slot = (fcnt + lane) & 31
                row = stage_v.at[slot]
                for g in range(DIM // 16):
                    vals = plsc.load_gather(
                        chunk_v,
                        [jnp.full((16,), k & 1, jnp.int32),
                         lanes + (16 * g),
                         jnp.full((16,), c, jnp.int32)],
                    )
                    row[pl.ds(16 * g, 16)] = vals
                pltpu.async_copy(
                    stage_v.at[pl.ds(slot, 1)], out_hbm.at[pl.ds(p, 1)], sem_o
                )
        return fcnt + rem, dcnt + ndrain

    def extract_tail_vec(e, nc, fcnt, dcnt):
        rem = jnp.minimum(nc - e * 16, 16)
        cv = cbuf_v[pl.ds(e * 16, 16)]
        pv = pbuf_v[pl.ds(e * 16, 16)]
        for lane in range(16):
            @pl.when(lane < rem)
            def _():
                c = cv[lane]
                p = pv[lane]
                pltpu.async_copy(
                    tail_v.at[pl.ds(c, 1)], out_hbm.at[pl.ds(p, 1)], sem_o
                )
        return fcnt + rem, dcnt

    def chunk_body(k, st):
        @pl.when(k + 1 < Q_PER_W)
        def _():
            fire(k + 1)

        wait_chunk(k)
        q = q_base + k
        g0 = q * CHW

        def batch_body(t2, st2):
            def gather_hits(t, off):
                hv = hid_v[pl.ds(t * 16, 16)]
                pvv = hpos_v[pl.ds(t * 16, 16)]
                valid = (t * 16 + lanes) < nh
                m = valid & (hv >= g0) & (hv < g0 + CHW)
                plsc.store_compressed(
                    cbuf_v.at[pl.ds(off, 16)], hv - g0, mask=m)
                plsc.store_compressed(
                    pbuf_v.at[pl.ds(off, 16)], pvv, mask=m)
                return off + plsc.all_reduce_population_count(m)[0]

            nc = lax.fori_loop(
                t2 * 16, jnp.minimum(t2 * 16 + 16, (nh + 15) // 16),
                gather_hits, jnp.int32(0))

            def ex(e, st3):
                return lax.cond(
                    q == TAILQ,
                    lambda s: extract_tail_vec(e, nc, s[0], s[1]),
                    lambda s: extract_vec(e, nc, k, s[0], s[1]),
                    st3,
                )

            return lax.fori_loop(0, (nc + 15) // 16, ex, st2)

        nb = jnp.where(q < NQ, (nh + 255) // 256, 0)
        return lax.fori_loop(0, nb, batch_body, st)

    fire(0)
    fcnt, dcnt = lax.fori_loop(
        0, Q_PER_W, chunk_body, (jnp.int32(0), jnp.int32(0)))
    drain_out_rows(fcnt - dcnt)


def kernel(user_ids, long_pref_emb):
    tail = long_pref_emb[NUSERS - TAILW:]
    return _gather_kernel(user_ids.astype(jnp.int32), long_pref_emb.T, tail)


# speedup vs baseline: 1.2908x; 1.1857x over previous
import functools

import jax
import jax.numpy as jnp
from jax import lax
from jax.experimental import pallas as pl
from jax.experimental.pallas import tpu as pltpu
from jax.experimental.pallas import tpu_sc as plsc

BATCH = 16384
DIM = 64
NUSERS = 1000000
NC = 2
NS = 16
NW = NC * NS
CHW = 256
NQ = (NUSERS + CHW - 1) // CHW
Q_PER_W = (NQ + NW - 1) // NW
TAILQ = NQ - 1
TAILW = NUSERS - TAILQ * CHW
GS = 8
NG = (Q_PER_W + GS - 1) // GS
CAPG = 496

_mesh = plsc.VectorSubcoreMesh(core_axis_name="c", subcore_axis_name="s")


@functools.partial(
    pl.kernel,
    mesh=_mesh,
    out_type=jax.ShapeDtypeStruct((BATCH, DIM), jnp.float32),
    scratch_types=[
        pltpu.VMEM((BATCH,), jnp.int32),
        pltpu.VMEM((BATCH + 16,), jnp.int32),
        pltpu.VMEM((BATCH + 16,), jnp.int32),
        pltpu.VMEM((2, DIM, CHW), jnp.float32),
        pltpu.VMEM((528,), jnp.int32),
        pltpu.VMEM((528,), jnp.int32),
        pltpu.VMEM((528,), jnp.int32),
        pltpu.VMEM((528,), jnp.int32),
        pltpu.VMEM((32, DIM), jnp.float32),
        pltpu.VMEM((TAILW, DIM), jnp.float32),
        pltpu.SemaphoreType.DMA,
        pltpu.SemaphoreType.DMA,
    ],
    compiler_params=pltpu.CompilerParams(needs_layout_passes=False),
)
def _gather_kernel(idx_hbm, table_t_hbm, tail_hbm, out_hbm, ids_v, hid_v,
                   hpos_v, chunk_v, cbuf_v, pbuf_v, gid_v, gpos_v, stage_v,
                   tail_v, sem_c, sem_o):
    wid = lax.axis_index("s") * NC + lax.axis_index("c")
    lanes = lax.iota(jnp.int32, 16)
    q_base = wid * Q_PER_W
    lo = q_base * CHW
    hi = jnp.minimum(lo + Q_PER_W * CHW, NUSERS)

    pltpu.sync_copy(idx_hbm, ids_v)
    pltpu.sync_copy(tail_hbm, tail_v)

    def scan_body(t, off):
        v = ids_v[pl.ds(t * 16, 16)]
        m = (v >= lo) & (v < hi)
        plsc.store_compressed(hid_v.at[pl.ds(off, 16)], v, mask=m)
        plsc.store_compressed(hpos_v.at[pl.ds(off, 16)], t * 16 + lanes, mask=m)
        return off + plsc.all_reduce_population_count(m)[0]

    nh = lax.fori_loop(0, BATCH // 16, scan_body, jnp.int32(0))

    def fire(k):
        q = q_base + k

        @pl.when(q < TAILQ)
        def _():
            g0 = pl.multiple_of(q * CHW, 128)
            pltpu.async_copy(
                table_t_hbm.at[:, pl.ds(g0, CHW)], chunk_v.at[k & 1], sem_c
            )

    def wait_chunk(k):
        q = q_base + k

        @pl.when(q < TAILQ)
        def _():
            pltpu.make_async_copy(
                table_t_hbm.at[:, pl.ds(0, CHW)], chunk_v.at[0], sem_c
            ).wait()

    def drain_out_rows(n):
        def d(_, carry):
            pltpu.make_async_copy(
                stage_v.at[pl.ds(0, 1)], out_hbm.at[pl.ds(0, 1)], sem_o
            ).wait()
            return carry

        lax.fori_loop(0, n, d, jnp.int32(0))

    def extract_vec(e, nc, k, fcnt, dcnt):
        rem = jnp.minimum(nc - e * 16, 16)
        ndrain = jnp.maximum(fcnt + rem - dcnt - 32, 0)
        drain_out_rows(ndrain)
        cv = cbuf_v[pl.ds(e * 16, 16)]
        pv = pbuf_v[pl.ds(e * 16, 16)]
        for lane in range(16):
            @pl.when(lane < rem)
            def _():
                c = cv[lane]
                p = pv[lane]
                slot = (fcnt + lane) & 31
                row = stage_v.at[slot]
                for g in range(DIM // 16):
                    vals = plsc.load_gather(
                        chunk_v,
                        [jnp.full((16,), k & 1, jnp.int32),
                         lanes + (16 * g),
                         jnp.full((16,), c, jnp.int32)],
                    )
                    row[pl.ds(16 * g, 16)] = vals
                pltpu.async_copy(
                    stage_v.at[pl.ds(slot, 1)], out_hbm.at[pl.ds(p, 1)], sem_o
                )
        return fcnt + rem, dcnt + ndrain

    def extract_tail_vec(e, nc, fcnt, dcnt):
        rem = jnp.minimum(nc - e * 16, 16)
        cv = cbuf_v[pl.ds(e * 16, 16)]
        pv = pbuf_v[pl.ds(e * 16, 16)]
        for lane in range(16):
            @pl.when(lane < rem)
            def _():
                c = cv[lane]
                p = pv[lane]
                pltpu.async_copy(
                    tail_v.at[pl.ds(c, 1)], out_hbm.at[pl.ds(p, 1)], sem_o
                )
        return fcnt + rem, dcnt

    def run_extract(nc, q, k, st):
        def ex(e, st3):
            return lax.cond(
                q == TAILQ,
                lambda s: extract_tail_vec(e, nc, s[0], s[1]),
                lambda s: extract_vec(e, nc, k, s[0], s[1]),
                st3,
            )

        return lax.fori_loop(0, (nc + 15) // 16, ex, st)

    def chunk_body(k, gn, st):
        @pl.when(k + 1 < Q_PER_W)
        def _():
            fire(k + 1)

        wait_chunk(k)
        q = q_base + k
        g0 = q * CHW

        def fast(st2):
            def gather_hits(t, off):
                hv = gid_v[pl.ds(t * 16, 16)]
                pvv = gpos_v[pl.ds(t * 16, 16)]
                valid = (t * 16 + lanes) < gn
                m = valid & (hv >= g0) & (hv < g0 + CHW)
                plsc.store_compressed(
                    cbuf_v.at[pl.ds(off, 16)], hv - g0, mask=m)
                plsc.store_compressed(
                    pbuf_v.at[pl.ds(off, 16)], pvv, mask=m)
                return off + plsc.all_reduce_population_count(m)[0]

            nc = lax.fori_loop(0, (gn + 15) // 16, gather_hits, jnp.int32(0))
            return run_extract(nc, q, k, st2)

        def slow(st2):
            def batch_body(t2, st3):
                def gather_hits(t, off):
                    hv = hid_v[pl.ds(t * 16, 16)]
                    pvv = hpos_v[pl.ds(t * 16, 16)]
                    valid = (t * 16 + lanes) < nh
                    m = valid & (hv >= g0) & (hv < g0 + CHW)
                    plsc.store_compressed(
                        cbuf_v.at[pl.ds(off, 16)], hv - g0, mask=m)
                    plsc.store_compressed(
                        pbuf_v.at[pl.ds(off, 16)], pvv, mask=m)
                    return off + plsc.all_reduce_population_count(m)[0]

                nc = lax.fori_loop(
                    t2 * 16, jnp.minimum(t2 * 16 + 16, (nh + 15) // 16),
                    gather_hits, jnp.int32(0))
                return run_extract(nc, q, k, st3)

            nb = jnp.where(q < NQ, (nh + 255) // 256, 0)
            return lax.fori_loop(0, nb, batch_body, st2)

        return lax.cond(gn > CAPG, slow, fast, st)

    def group_body(m, st):
        glo = lo + m * GS * CHW
        ghi = glo + GS * CHW

        def compact(t, off):
            hv = hid_v[pl.ds(t * 16, 16)]
            pvv = hpos_v[pl.ds(t * 16, 16)]
            valid = (t * 16 + lanes) < nh
            m2 = valid & (hv >= glo) & (hv < ghi)
            soff = jnp.minimum(off, CAPG)
            plsc.store_compressed(gid_v.at[pl.ds(soff, 16)], hv, mask=m2)
            plsc.store_compressed(gpos_v.at[pl.ds(soff, 16)], pvv, mask=m2)
            return off + plsc.all_reduce_population_count(m2)[0]

        gn = lax.fori_loop(0, (nh + 15) // 16, compact, jnp.int32(0))

        def kk_body(kk, st2):
            return chunk_body(m * GS + kk, gn, st2)

        return lax.fori_loop(
            0, jnp.minimum(GS, Q_PER_W - m * GS), kk_body, st)

    fire(0)
    fcnt, dcnt = lax.fori_loop(
        0, NG, group_body, (jnp.int32(0), jnp.int32(0)))
    drain_out_rows(fcnt - dcnt)


def kernel(user_ids, long_pref_emb):
    tail = long_pref_emb[NUSERS - TAILW:]
    return _gather_kernel(user_ids.astype(jnp.int32), long_pref_emb.T, tail)


# speedup vs baseline: 1.2910x; 1.0001x over previous
import functools

import jax
import jax.numpy as jnp
from jax import lax
from jax.experimental import pallas as pl
from jax.experimental.pallas import tpu as pltpu
from jax.experimental.pallas import tpu_sc as plsc

BATCH = 16384
DIM = 64
NUSERS = 1000000
NC = 2
NS = 16
NW = NC * NS
CHW = 256
NQ = (NUSERS + CHW - 1) // CHW
Q_PER_W = (NQ + NW - 1) // NW
TAILQ = NQ - 1
TAILW = NUSERS - TAILQ * CHW
GS = 8
NG = (Q_PER_W + GS - 1) // GS
CAPG = 496

_mesh = plsc.VectorSubcoreMesh(core_axis_name="c", subcore_axis_name="s")


@functools.partial(
    pl.kernel,
    mesh=_mesh,
    out_type=jax.ShapeDtypeStruct((BATCH, DIM), jnp.float32),
    scratch_types=[
        pltpu.VMEM((BATCH,), jnp.int32),
        pltpu.VMEM((BATCH + 16,), jnp.int32),
        pltpu.VMEM((BATCH + 16,), jnp.int32),
        pltpu.VMEM((2, DIM, CHW), jnp.float32),
        pltpu.VMEM((528,), jnp.int32),
        pltpu.VMEM((528,), jnp.int32),
        pltpu.VMEM((528,), jnp.int32),
        pltpu.VMEM((528,), jnp.int32),
        pltpu.VMEM((32, DIM), jnp.float32),
        pltpu.VMEM((TAILW, DIM), jnp.float32),
        pltpu.SemaphoreType.DMA,
        pltpu.SemaphoreType.DMA,
    ],
    compiler_params=pltpu.CompilerParams(needs_layout_passes=False),
)
def _gather_kernel(idx_hbm, table_t_hbm, tail_hbm, out_hbm, ids_v, hid_v,
                   hpos_v, chunk_v, cbuf_v, pbuf_v, gid_v, gpos_v, stage_v,
                   tail_v, sem_c, sem_o):
    wid = lax.axis_index("s") * NC + lax.axis_index("c")
    lanes = lax.iota(jnp.int32, 16)
    q_base = wid * Q_PER_W
    lo = q_base * CHW
    hi = jnp.minimum(lo + Q_PER_W * CHW, NUSERS)

    pltpu.sync_copy(idx_hbm, ids_v)
    pltpu.sync_copy(tail_hbm, tail_v)

    def scan_body(t, off):
        v = ids_v[pl.ds(t * 16, 16)]
        m = (v >= lo) & (v < hi)
        plsc.store_compressed(hid_v.at[pl.ds(off, 16)], v, mask=m)
        plsc.store_compressed(hpos_v.at[pl.ds(off, 16)], t * 16 + lanes, mask=m)
        return off + plsc.all_reduce_population_count(m)[0]

    nh = lax.fori_loop(0, BATCH // 16, scan_body, jnp.int32(0), unroll=4)

    def fire(k):
        q = q_base + k

        @pl.when(q < TAILQ)
        def _():
            g0 = pl.multiple_of(q * CHW, 128)
            pltpu.async_copy(
                table_t_hbm.at[:, pl.ds(g0, CHW)], chunk_v.at[k & 1], sem_c
            )

    def wait_chunk(k):
        q = q_base + k

        @pl.when(q < TAILQ)
        def _():
            pltpu.make_async_copy(
                table_t_hbm.at[:, pl.ds(0, CHW)], chunk_v.at[0], sem_c
            ).wait()

    def drain_out_rows(n):
        def d(_, carry):
            pltpu.make_async_copy(
                stage_v.at[pl.ds(0, 1)], out_hbm.at[pl.ds(0, 1)], sem_o
            ).wait()
            return carry

        lax.fori_loop(0, n, d, jnp.int32(0))

    def extract_vec(e, nc, k, fcnt, dcnt):
        rem = jnp.minimum(nc - e * 16, 16)
        ndrain = jnp.maximum(fcnt + rem - dcnt - 32, 0)
        drain_out_rows(ndrain)
        cv = cbuf_v[pl.ds(e * 16, 16)]
        pv = pbuf_v[pl.ds(e * 16, 16)]
        for lane in range(16):
            @pl.when(lane < rem)
            def _():
                c = cv[lane]
                p = pv[lane]
                slot = (fcnt + lane) & 31
                row = stage_v.at[slot]
                for g in range(DIM // 16):
                    vals = plsc.load_gather(
                        chunk_v,
                        [jnp.full((16,), k & 1, jnp.int32),
                         lanes + (16 * g),
                         jnp.full((16,), c, jnp.int32)],
                    )
                    row[pl.ds(16 * g, 16)] = vals
                pltpu.async_copy(
                    stage_v.at[pl.ds(slot, 1)], out_hbm.at[pl.ds(p, 1)], sem_o
                )
        return fcnt + rem, dcnt + ndrain

    def extract_tail_vec(e, nc, fcnt, dcnt):
        rem = jnp.minimum(nc - e * 16, 16)
        cv = cbuf_v[pl.ds(e * 16, 16)]
        pv = pbuf_v[pl.ds(e * 16, 16)]
        for lane in range(16):
            @pl.when(lane < rem)
            def _():
                c = cv[lane]
                p = pv[lane]
                pltpu.async_copy(
                    tail_v.at[pl.ds(c, 1)], out_hbm.at[pl.ds(p, 1)], sem_o
                )
        return fcnt + rem, dcnt

    def run_extract(nc, q, k, st):
        def ex(e, st3):
            return lax.cond(
                q == TAILQ,
                lambda s: extract_tail_vec(e, nc, s[0], s[1]),
                lambda s: extract_vec(e, nc, k, s[0], s[1]),
                st3,
            )

        return lax.fori_loop(0, (nc + 15) // 16, ex, st)

    def chunk_body(k, gn, st):
        @pl.when(k + 1 < Q_PER_W)
        def _():
            fire(k + 1)

        wait_chunk(k)
        q = q_base + k
        g0 = q * CHW

        def fast(st2):
            def gather_hits(t, off):
                hv = gid_v[pl.ds(t * 16, 16)]
                pvv = gpos_v[pl.ds(t * 16, 16)]
                valid = (t * 16 + lanes) < gn
                m = valid & (hv >= g0) & (hv < g0 + CHW)
                plsc.store_compressed(
                    cbuf_v.at[pl.ds(off, 16)], hv - g0, mask=m)
                plsc.store_compressed(
                    pbuf_v.at[pl.ds(off, 16)], pvv, mask=m)
                return off + plsc.all_reduce_population_count(m)[0]

            nc = lax.fori_loop(0, (gn + 15) // 16, gather_hits, jnp.int32(0))
            return run_extract(nc, q, k, st2)

        def slow(st2):
            def batch_body(t2, st3):
                def gather_hits(t, off):
                    hv = hid_v[pl.ds(t * 16, 16)]
                    pvv = hpos_v[pl.ds(t * 16, 16)]
                    valid = (t * 16 + lanes) < nh
                    m = valid & (hv >= g0) & (hv < g0 + CHW)
                    plsc.store_compressed(
                        cbuf_v.at[pl.ds(off, 16)], hv - g0, mask=m)
                    plsc.store_compressed(
                        pbuf_v.at[pl.ds(off, 16)], pvv, mask=m)
                    return off + plsc.all_reduce_population_count(m)[0]

                nc = lax.fori_loop(
                    t2 * 16, jnp.minimum(t2 * 16 + 16, (nh + 15) // 16),
                    gather_hits, jnp.int32(0))
                return run_extract(nc, q, k, st3)

            nb = jnp.where(q < NQ, (nh + 255) // 256, 0)
            return lax.fori_loop(0, nb, batch_body, st2)

        return lax.cond(gn > CAPG, slow, fast, st)

    def group_body(m, st):
        glo = lo + m * GS * CHW
        ghi = glo + GS * CHW

        def compact(t, off):
            hv = hid_v[pl.ds(t * 16, 16)]
            pvv = hpos_v[pl.ds(t * 16, 16)]
            valid = (t * 16 + lanes) < nh
            m2 = valid & (hv >= glo) & (hv < ghi)
            soff = jnp.minimum(off, CAPG)
            plsc.store_compressed(gid_v.at[pl.ds(soff, 16)], hv, mask=m2)
            plsc.store_compressed(gpos_v.at[pl.ds(soff, 16)], pvv, mask=m2)
            return off + plsc.all_reduce_population_count(m2)[0]

        gn = lax.fori_loop(0, (nh + 15) // 16, compact, jnp.int32(0))

        def kk_body(kk, st2):
            return chunk_body(m * GS + kk, gn, st2)

        return lax.fori_loop(
            0, jnp.minimum(GS, Q_PER_W - m * GS), kk_body, st)

    fire(0)
    fcnt, dcnt = lax.fori_loop(
        0, NG, group_body, (jnp.int32(0), jnp.int32(0)))
    drain_out_rows(fcnt - dcnt)


def kernel(user_ids, long_pref_emb):
    tail = long_pref_emb[NUSERS - TAILW:]
    return _gather_kernel(user_ids.astype(jnp.int32), long_pref_emb.T, tail)
